# baseline (speedup 1.0000x reference)
import functools

import jax
import jax.numpy as jnp
from jax import lax
from jax.experimental import pallas as pl
from jax.experimental.pallas import tpu as pltpu
from jax.experimental.pallas import tpu_sc as plsc

_C = 64


def _gather_kernel(N, D, V, NW, per_w, C, nch):
    mesh = plsc.VectorSubcoreMesh(core_axis_name="c", subcore_axis_name="s")
    T = nch // 4

    @functools.partial(
        pl.kernel,
        mesh=mesh,
        compiler_params=pltpu.CompilerParams(needs_layout_passes=False),
        out_type=jax.ShapeDtypeStruct((N, D), jnp.float32),
        scratch_types=[
            pltpu.VMEM((nch, C), jnp.int32),
            pltpu.VMEM_SHARED((V, D), jnp.float32),
            pltpu.VMEM((V, D), jnp.float32),
            pltpu.VMEM((C, D), jnp.float32),
            pltpu.VMEM((C, D), jnp.float32),
            pltpu.VMEM((C, D), jnp.float32),
            pltpu.VMEM((C, D), jnp.float32),
            pltpu.VMEM((C,), jnp.int32),
            pltpu.SemaphoreType.DMA,
            pltpu.SemaphoreType.DMA,
            pltpu.SemaphoreType.DMA,
            pltpu.SemaphoreType.DMA,
            pltpu.SemaphoreType.DMA,
            pltpu.SemaphoreType.DMA,
        ],
    )
    def k(idx_hbm, table_hbm, out_hbm, idx_v, table_sh, table_v,
          sbuf0, sbuf1, vbuf0, vbuf1, idx_row,
          gsem0, gsem1, ss0, ss1, sv0, sv1):
        sid = lax.axis_index("s")
        wid = sid * 2 + lax.axis_index("c")
        base = wid * per_w

        @pl.when(sid == 0)
        def _stage_table_sh():
            pltpu.sync_copy(table_hbm, table_sh)

        pltpu.sync_copy(table_hbm, table_v)
        pltpu.sync_copy(idx_hbm.at[wid], idx_v)
        plsc.subcore_barrier()

        iota16 = lax.iota(jnp.int32, 16)

        pltpu.async_copy(table_sh.at[idx_v.at[0]], sbuf0, gsem0)
        pltpu.async_copy(table_sh.at[idx_v.at[1]], sbuf1, gsem1)

        def fill(buf, g):
            for rg in range(C // 16):
                rows16 = rg * 16 + iota16
                iv = idx_v[g, pl.ds(rg * 16, 16)]

                def cbody(c8, _, rows16=rows16, iv=iv):
                    for u in range(8):
                        cc = c8 * 8 + u
                        colv = jnp.broadcast_to(cc, (16,))
                        vals = plsc.load_gather(table_v, [iv, colv])
                        plsc.store_scatter(buf, [rows16, colv], vals)
                    return 0

                lax.fori_loop(0, D // 8, cbody, 0)

        def body(t, _):
            c0 = t * 4

            @pl.when(t > 0)
            def _w0():
                pltpu.make_async_copy(vbuf0, out_hbm.at[pl.ds(base, C)],
                                      sv0).wait()

            fill(vbuf0, c0 + 2)
            pltpu.async_copy(vbuf0, out_hbm.at[pl.ds(base + (c0 + 2) * C, C)],
                             sv0)

            @pl.when(t > 0)
            def _w1():
                pltpu.make_async_copy(vbuf1, out_hbm.at[pl.ds(base, C)],
                                      sv1).wait()

            fill(vbuf1, c0 + 3)
            pltpu.async_copy(vbuf1, out_hbm.at[pl.ds(base + (c0 + 3) * C, C)],
                             sv1)

            pltpu.make_async_copy(table_sh.at[idx_v.at[0]], sbuf0,
                                  gsem0).wait()
            pltpu.async_copy(sbuf0, out_hbm.at[pl.ds(base + c0 * C, C)], ss0)
            pltpu.make_async_copy(table_sh.at[idx_v.at[0]], sbuf1,
                                  gsem1).wait()
            pltpu.async_copy(sbuf1, out_hbm.at[pl.ds(base + (c0 + 1) * C, C)],
                             ss1)

            pltpu.make_async_copy(sbuf0, out_hbm.at[pl.ds(base, C)], ss0).wait()

            @pl.when(t < T - 1)
            def _g0():
                pltpu.async_copy(table_sh.at[idx_v.at[c0 + 4]], sbuf0, gsem0)

            pltpu.make_async_copy(sbuf1, out_hbm.at[pl.ds(base, C)], ss1).wait()

            @pl.when(t < T - 1)
            def _g1():
                pltpu.async_copy(table_sh.at[idx_v.at[c0 + 5]], sbuf1, gsem1)

            return 0

        lax.fori_loop(0, T, body, 0)

        pltpu.make_async_copy(vbuf0, out_hbm.at[pl.ds(base, C)], sv0).wait()
        pltpu.make_async_copy(vbuf1, out_hbm.at[pl.ds(base, C)], sv1).wait()

    return k


def kernel(categories, ce):
    B, S = categories.shape
    V, D = ce.shape
    N = B * S
    NW = 32
    per_w = N // NW
    C = _C
    nch = per_w // C
    idx3 = categories.reshape(NW, nch, C)
    out = _gather_kernel(N, D, V, NW, per_w, C, nch)(idx3, ce)
    return out.reshape(B, S, D)


# speedup vs baseline: 10.6040x; 10.6040x over previous
import functools

import jax
import jax.numpy as jnp
from jax import lax
from jax.experimental import pallas as pl
from jax.experimental.pallas import tpu as pltpu
from jax.experimental.pallas import tpu_sc as plsc

_NBUF = 4
_CHUNK = 128


def _gather_kernel(N, D, V, NW, per_w, C, nch):
    mesh = plsc.VectorSubcoreMesh(core_axis_name="c", subcore_axis_name="s")
    NBUF = _NBUF
    T = nch // NBUF

    @functools.partial(
        pl.kernel,
        mesh=mesh,
        out_type=jax.ShapeDtypeStruct((N, D), jnp.float32),
        scratch_types=[
            pltpu.VMEM((nch, C), jnp.int32),
            pltpu.VMEM_SHARED((V, D), jnp.float32),
            pltpu.VMEM((2, 2 * C, D), jnp.float32),
            pltpu.SemaphoreType.DMA,
            pltpu.SemaphoreType.DMA,
            pltpu.SemaphoreType.DMA,
            pltpu.SemaphoreType.DMA,
            pltpu.SemaphoreType.DMA,
            pltpu.SemaphoreType.DMA,
        ],
    )
    def k(idx_hbm, table_hbm, out_hbm, idx_v, table_sh, rows_v,
          g0, g1, g2, g3, ss0, ss1):
        gsem = (g0, g1, g2, g3)
        sid = lax.axis_index("s")
        wid = sid * 2 + lax.axis_index("c")
        base = wid * per_w

        @pl.when(sid == 0)
        def _stage_table():
            pltpu.sync_copy(table_hbm, table_sh)

        pltpu.sync_copy(idx_hbm.at[wid], idx_v)
        plsc.subcore_barrier()

        for b in range(2):
            for h in range(2):
                pltpu.async_copy(table_sh.at[idx_v.at[2 * b + h]],
                                 rows_v.at[b].at[pl.ds(h * C, C)],
                                 gsem[2 * b + h])

        def body(t, _):
            c0 = t * NBUF

            for h in range(2):
                pltpu.make_async_copy(
                    table_sh.at[idx_v.at[0]],
                    rows_v.at[0].at[pl.ds(h * C, C)], gsem[h]).wait()
            pltpu.async_copy(
                rows_v.at[0], out_hbm.at[pl.ds(base + c0 * C, 2 * C)], ss0)

            for h in range(2):
                pltpu.make_async_copy(
                    table_sh.at[idx_v.at[0]],
                    rows_v.at[1].at[pl.ds(h * C, C)], gsem[2 + h]).wait()
            pltpu.async_copy(
                rows_v.at[1], out_hbm.at[pl.ds(base + (c0 + 2) * C, 2 * C)],
                ss1)

            pltpu.make_async_copy(
                rows_v.at[0], out_hbm.at[pl.ds(base, 2 * C)], ss0).wait()

            @pl.when(t < T - 1)
            def _g01():
                for h in range(2):
                    pltpu.async_copy(
                        table_sh.at[idx_v.at[c0 + 4 + h]],
                        rows_v.at[0].at[pl.ds(h * C, C)], gsem[h])

            pltpu.make_async_copy(
                rows_v.at[1], out_hbm.at[pl.ds(base, 2 * C)], ss1).wait()

            @pl.when(t < T - 1)
            def _g23():
                for h in range(2):
                    pltpu.async_copy(
                        table_sh.at[idx_v.at[c0 + 6 + h]],
                        rows_v.at[1].at[pl.ds(h * C, C)], gsem[2 + h])

            return 0

        lax.fori_loop(0, T, body, 0)

    return k


def kernel(categories, ce):
    B, S = categories.shape
    V, D = ce.shape
    N = B * S
    NW = 32
    per_w = N // NW
    C = _CHUNK
    nch = per_w // C
    idx3 = categories.reshape(NW, nch, C)
    out = _gather_kernel(N, D, V, NW, per_w, C, nch)(idx3, ce)
    return out.reshape(B, S, D)


# speedup vs baseline: 10.6318x; 1.0026x over previous
import functools

import jax
import jax.numpy as jnp
from jax import lax
from jax.experimental import pallas as pl
from jax.experimental.pallas import tpu as pltpu
from jax.experimental.pallas import tpu_sc as plsc

_NBUF = 8
_CHUNK = 64


def _gather_kernel(N, D, V, NW, per_w, C, nch):
    mesh = plsc.VectorSubcoreMesh(core_axis_name="c", subcore_axis_name="s")
    NBUF = _NBUF
    ngroups = nch // NBUF

    sem_types = [pltpu.SemaphoreType.DMA] * (2 * NBUF)

    @functools.partial(
        pl.kernel,
        mesh=mesh,
        out_type=jax.ShapeDtypeStruct((N, D), jnp.float32),
        scratch_types=[
            pltpu.VMEM((nch, C), jnp.int32),
            pltpu.VMEM_SHARED((V, D), jnp.float32),
            pltpu.VMEM((NBUF, C, D), jnp.float32),
        ]
        + sem_types,
    )
    def k(idx_hbm, table_hbm, out_hbm, idx_v, table_sh, rows_v,
          g0, g1, g2, g3, g4, g5, g6, g7,
          s0, s1, s2, s3, s4, s5, s6, s7):
        gsem = (g0, g1, g2, g3, g4, g5, g6, g7)
        ssem = (s0, s1, s2, s3, s4, s5, s6, s7)
        sid = lax.axis_index("s")
        wid = sid * 2 + lax.axis_index("c")
        base = wid * per_w

        @pl.when(sid == 0)
        def _stage_table():
            pltpu.sync_copy(table_hbm, table_sh)

        pltpu.sync_copy(idx_hbm.at[wid], idx_v)
        plsc.subcore_barrier()

        for b in range(NBUF):
            pltpu.async_copy(table_sh.at[idx_v.at[b]], rows_v.at[b], gsem[b])

        def body(t, _):
            c0 = t * NBUF
            for b in range(NBUF):
                pltpu.make_async_copy(
                    table_sh.at[idx_v.at[0]], rows_v.at[b], gsem[b]).wait()
                pltpu.async_copy(
                    rows_v.at[b], out_hbm.at[pl.ds(base + (c0 + b) * C, C)],
                    ssem[b])
            for b in range(NBUF):
                pltpu.make_async_copy(
                    rows_v.at[b], out_hbm.at[pl.ds(base, C)], ssem[b]).wait()

                @pl.when(t < ngroups - 1)
                def _issue(b=b, c0=c0):
                    pltpu.async_copy(
                        table_sh.at[idx_v.at[c0 + NBUF + b]], rows_v.at[b],
                        gsem[b])

            return 0

        lax.fori_loop(0, ngroups, body, 0)

    return k


def kernel(categories, ce):
    B, S = categories.shape
    V, D = ce.shape
    N = B * S
    NW = 32
    per_w = N // NW
    C = _CHUNK
    nch = per_w // C
    idx3 = categories.reshape(NW, nch, C)
    out = _gather_kernel(N, D, V, NW, per_w, C, nch)(idx3, ce)
    return out.reshape(B, S, D)
